# baseline (speedup 1.0000x reference)
import functools

import jax
import jax.numpy as jnp
from jax import lax
from jax.experimental import pallas as pl
from jax.experimental.pallas import tpu as pltpu
from jax.experimental.pallas import tpu_sc as plsc

_K = 128
_L = 16


def _lane_shuffle(x, idx_lanes):
    dnums = lax.GatherDimensionNumbers(
        offset_dims=(), collapsed_slice_dims=(0,), start_index_map=(0,))
    return lax.gather(x, idx_lanes[:, None], dnums, (1,),
                      mode=lax.GatherScatterMode.PROMISE_IN_BOUNDS)


def _glove_body(idx_hbm, table_hbm, bvec_hbm, out_hbm,
                idx_v, rows_v, bv_v, out_v, sem_w, sem_b):
    cid = lax.axis_index("c")
    sid = lax.axis_index("s")

    @pl.when(jnp.logical_and(cid == 0, sid == 0))
    def _():
        pltpu.sync_copy(idx_hbm, idx_v)
        cp_w = pltpu.async_copy(table_hbm.at[idx_v], rows_v, sem_w)
        cp_b = pltpu.async_copy(bvec_hbm.at[idx_v], bv_v, sem_b)
        cp_w.wait()
        cp_b.wait()
        lane = lax.iota(jnp.int32, _L)
        acc = jnp.where(lane < 2, bv_v[...], 0.0)
        for c in range(_K // _L):
            acc = acc + (rows_v[0, pl.ds(c * _L, _L)]
                         * rows_v[1, pl.ds(c * _L, _L)])
        for sh in (8, 4, 2, 1):
            acc = acc + _lane_shuffle(acc, (lane + sh) & (_L - 1))
        out_v[...] = acc
        pltpu.sync_copy(out_v, out_hbm)


_glove_sc = functools.partial(
    pl.kernel,
    out_type=jax.ShapeDtypeStruct((_L,), jnp.float32),
    mesh=plsc.VectorSubcoreMesh(core_axis_name="c", subcore_axis_name="s"),
    scratch_types=[
        pltpu.VMEM((_L,), jnp.int32),
        pltpu.VMEM((_L, _K), jnp.float32),
        pltpu.VMEM((_L,), jnp.float32),
        pltpu.VMEM((_L,), jnp.float32),
        pltpu.SemaphoreType.DMA,
        pltpu.SemaphoreType.DMA,
    ],
)(_glove_body)


def kernel(W, B, i, j):
    ii = jnp.asarray(i, jnp.int32)
    jj = jnp.asarray(j, jnp.int32)
    idx = jnp.full((_L,), ii, dtype=jnp.int32).at[1].set(jj)
    out = _glove_sc(idx, W, B)
    return out[0]


# speedup vs baseline: 1.0679x; 1.0679x over previous
import functools

import jax
import jax.numpy as jnp
from jax import lax
from jax.experimental import pallas as pl
from jax.experimental.pallas import tpu as pltpu
from jax.experimental.pallas import tpu_sc as plsc

_K = 128
_L = 16


def _lane_shuffle(x, idx_lanes):
    dnums = lax.GatherDimensionNumbers(
        offset_dims=(), collapsed_slice_dims=(0,), start_index_map=(0,))
    return lax.gather(x, idx_lanes[:, None], dnums, (1,),
                      mode=lax.GatherScatterMode.PROMISE_IN_BOUNDS)


def _glove_body(idx_hbm, table_hbm, bvec_hbm, out_hbm,
                idx_v, rows_v, bv_v, out_v, sem_w, sem_b):
    cid = lax.axis_index("c")
    sid = lax.axis_index("s")

    @pl.when(jnp.logical_and(cid == 0, sid == 0))
    def _():
        pltpu.sync_copy(idx_hbm, idx_v)
        cp_w = pltpu.async_copy(table_hbm.at[idx_v], rows_v, sem_w)
        cp_b = pltpu.async_copy(bvec_hbm.at[idx_v], bv_v, sem_b)
        cp_w.wait()
        cp_b.wait()
        lane = lax.iota(jnp.int32, _L)
        acc = jnp.where(lane < 2, bv_v[...], 0.0)
        for c in range(_K // _L):
            acc = acc + (rows_v[0, pl.ds(c * _L, _L)]
                         * rows_v[1, pl.ds(c * _L, _L)])
        for sh in (8, 4, 2, 1):
            acc = acc + _lane_shuffle(acc, (lane + sh) & (_L - 1))
        out_v[...] = acc
        pltpu.sync_copy(out_v, out_hbm)


_glove_sc = functools.partial(
    pl.kernel,
    out_type=jax.ShapeDtypeStruct((_L,), jnp.float32),
    mesh=plsc.VectorSubcoreMesh(core_axis_name="c", subcore_axis_name="s",
                                num_cores=1, num_subcores=1),
    scratch_types=[
        pltpu.VMEM((_L,), jnp.int32),
        pltpu.VMEM((_L, _K), jnp.float32),
        pltpu.VMEM((_L,), jnp.float32),
        pltpu.VMEM((_L,), jnp.float32),
        pltpu.SemaphoreType.DMA,
        pltpu.SemaphoreType.DMA,
    ],
)(_glove_body)


def kernel(W, B, i, j):
    ii = jnp.asarray(i, jnp.int32)
    jj = jnp.asarray(j, jnp.int32)
    idx = jnp.full((_L,), ii, dtype=jnp.int32).at[1].set(jj)
    out = _glove_sc(idx, W, B)
    return out[0]
